# baseline (speedup 1.0000x reference)
import jax
import jax.numpy as jnp
from jax import lax
from jax.experimental import pallas as pl
from jax.experimental.pallas import tpu as pltpu
from jax.experimental.pallas import tpu_sc as plsc

N_NODES = 10000
N_EDGES = 10000
N_INC = 320000
F = 128
FH = F // 2

NC = 2
NS = 16
CHUNK = 125
ROWS_TOTAL = N_INC // CHUNK
ROWS_PER_TILE = ROWS_TOTAL // NS
BLK = 16
NBLK = ROWS_PER_TILE // BLK

SC_STRIPE = 640
SC_LAST_OFF = (NS - 1) * SC_STRIPE
SC_LAST = N_NODES - SC_LAST_OFF

_mesh = plsc.VectorSubcoreMesh(core_axis_name="c", subcore_axis_name="s")


@pl.kernel(
    out_type=(jax.ShapeDtypeStruct((N_NODES, FH), jnp.float32),
              jax.ShapeDtypeStruct((N_NODES, FH), jnp.float32),
              jax.ShapeDtypeStruct((N_NODES, FH), jnp.float32),
              jax.ShapeDtypeStruct((N_NODES, FH), jnp.float32)),
    mesh=_mesh,
    scratch_types=[
        pltpu.VMEM((2, BLK, CHUNK), jnp.int32),
        pltpu.VMEM((2, BLK, CHUNK), jnp.int32),
        pltpu.VMEM((2, CHUNK, FH), jnp.float32),
        pltpu.VMEM((BLK, CHUNK), jnp.float32),
        pltpu.VMEM((CHUNK,), jnp.float32),
        pltpu.VMEM((SC_STRIPE,), jnp.float32),
        pltpu.VMEM((SC_STRIPE + 16,), jnp.float32),
        pltpu.VMEM((128, FH), jnp.float32),
        pltpu.VMEM((1, FH), jnp.float32),
        pltpu.VMEM_SHARED((N_NODES, FH), jnp.float32),
        pltpu.VMEM_SHARED((N_NODES,), jnp.float32),
        pltpu.VMEM_SHARED((N_NODES,), jnp.float32),
        pltpu.SemaphoreType.DMA,
        pltpu.SemaphoreType.DMA,
        pltpu.SemaphoreType.DMA,
        pltpu.SemaphoreType.DMA,
        pltpu.SemaphoreType.DMA,
    ],
    compiler_params=pltpu.CompilerParams(use_tc_tiling_on_sc=False),
)
def _mega_kernel(nidx_hbm, eidx_hbm, xlo_hbm, xhi_hbm, w_hbm,
                 zeros1_hbm, zeros2_hbm, bias_lo_hbm, bias_hi_hbm,
                 ylo_hbm, yhi_hbm, eolo_hbm, eohi_hbm,
                 nblk_v, eblk_v, rows_v, wval_v, ones_v,
                 deg_v, inv_v, scl_v, bias_v,
                 acc_sh, dacc_sh, bacc_sh,
                 semg, semi, semw, semd, semb):
    cid = lax.axis_index("c")
    sid = lax.axis_index("s")

    def for_stripe_blocks(fn):
        @pl.when(sid < NS - 1)
        def _main():
            for k in range(SC_STRIPE // 128):
                off = pl.multiple_of(sid * SC_STRIPE + k * 128, 8)
                fn(off, k * 128, 128)

        @pl.when(sid == NS - 1)
        def _last():
            for k in range(SC_LAST // 128):
                fn(SC_LAST_OFF + k * 128, k * 128, 128)
            rem = SC_LAST % 128
            if rem:
                fn(SC_LAST_OFF + (SC_LAST // 128) * 128,
                   (SC_LAST // 128) * 128, rem)

    def zero_acc():
        for_stripe_blocks(lambda goff, loff, sz: pltpu.sync_copy(
            zeros2_hbm.at[pl.ds(goff, sz)], acc_sh.at[pl.ds(goff, sz)]))

    zero_acc()

    @pl.when(sid == 0)
    def _zero_d():
        pltpu.sync_copy(zeros1_hbm, dacc_sh)

    @pl.when(sid == 1)
    def _zero_bdeg():
        pltpu.sync_copy(zeros1_hbm, bacc_sh)

    for i in range((CHUNK + 15) // 16):
        ones_v[pl.ds(min(16 * i, CHUNK - 16), 16)] = jnp.full(
            (16,), 1.0, jnp.float32)

    @pl.when(cid == 0)
    def _stage_bias_lo():
        pltpu.sync_copy(bias_lo_hbm, bias_v)

    @pl.when(cid == 1)
    def _stage_bias_hi():
        pltpu.sync_copy(bias_hi_hbm, bias_v)

    def run_phase(gidx_hbm, gblk_v, sidx_hbm, sblk_v, tbl_lo, tbl_hi,
                  with_degrees):
        def fire_gather(pb, j, buf):
            @pl.when(cid == 0)
            def _lo():
                pltpu.async_copy(tbl_lo.at[gblk_v.at[pb, j]],
                                 rows_v.at[buf], semg)

            @pl.when(cid == 1)
            def _hi():
                pltpu.async_copy(tbl_hi.at[gblk_v.at[pb, j]],
                                 rows_v.at[buf], semg)

        pltpu.async_copy(gidx_hbm.at[sid, pl.ds(0, BLK)], gblk_v.at[0], semi)
        pltpu.async_copy(sidx_hbm.at[sid, pl.ds(0, BLK)], sblk_v.at[0], semi)

        def outer(b, carry):
            pb = b % 2
            pltpu.make_async_copy(gidx_hbm.at[sid, pl.ds(0, BLK)],
                                  gblk_v.at[pb], semi).wait()
            pltpu.make_async_copy(sidx_hbm.at[sid, pl.ds(0, BLK)],
                                  sblk_v.at[pb], semi).wait()

            @pl.when(b < NBLK - 1)
            def _fire_next_block():
                off = pl.multiple_of((b + 1) * BLK, 8)
                pltpu.async_copy(gidx_hbm.at[sid, pl.ds(off, BLK)],
                                 gblk_v.at[(b + 1) % 2], semi)
                pltpu.async_copy(sidx_hbm.at[sid, pl.ds(off, BLK)],
                                 sblk_v.at[(b + 1) % 2], semi)

            fire_gather(pb, 0, 0)
            if with_degrees:
                pltpu.async_copy(w_hbm.at[sblk_v.at[pb, 0]], wval_v.at[0],
                                 semw)

            def inner(j, c2):
                @pl.when(j < BLK - 1)
                def _fire_next():
                    fire_gather(pb, j + 1, (j + 1) % 2)
                    if with_degrees:
                        pltpu.async_copy(w_hbm.at[sblk_v.at[pb, j + 1]],
                                         wval_v.at[j + 1], semw)
                pltpu.make_async_copy(tbl_lo.at[gblk_v.at[pb, 0]],
                                      rows_v.at[j % 2], semg).wait()
                pltpu.sync_copy(rows_v.at[j % 2],
                                acc_sh.at[sblk_v.at[pb, j]], add=True)
                if with_degrees:
                    pltpu.make_async_copy(w_hbm.at[sblk_v.at[pb, 0]],
                                          wval_v.at[0], semw).wait()
                    pltpu.async_copy(wval_v.at[j],
                                     dacc_sh.at[gblk_v.at[pb, j]], semd,
                                     add=True)
                    pltpu.async_copy(ones_v, bacc_sh.at[sblk_v.at[pb, j]],
                                     semb, add=True)
                return c2
            lax.fori_loop(0, BLK, inner, 0)

            if with_degrees:
                def draind(j, c3):
                    pltpu.make_async_copy(
                        wval_v.at[0], dacc_sh.at[gblk_v.at[0, 0]],
                        semd).wait()
                    return c3
                lax.fori_loop(0, BLK, draind, 0)
            return carry
        lax.fori_loop(0, NBLK, outer, 0)

        if with_degrees:
            def drainb(j, c4):
                pltpu.make_async_copy(ones_v, bacc_sh.at[sblk_v.at[0, 0]],
                                      semb).wait()
                return c4
            lax.fori_loop(0, ROWS_PER_TILE, drainb, 0)

    def compute_inverse(src_sh):
        @pl.when(sid < NS - 1)
        def _main():
            off = pl.multiple_of(sid * SC_STRIPE, 8)
            pltpu.sync_copy(src_sh.at[pl.ds(off, SC_STRIPE)],
                            deg_v.at[pl.ds(0, SC_STRIPE)])

        @pl.when(sid == NS - 1)
        def _last():
            pltpu.sync_copy(src_sh.at[pl.ds(SC_LAST_OFF, SC_LAST)],
                            deg_v.at[pl.ds(0, SC_LAST)])

        def inv_group(i, carry):
            g = deg_v[pl.ds(i * 16, 16)]
            inv_v[pl.ds(i * 16, 16)] = jnp.where(
                g > 0, 1.0 / jnp.where(g > 0, g, 1.0), 0.0)
            return carry
        lax.fori_loop(0, SC_STRIPE // 16, inv_group, 0)

    def scale_and_write(out_lo, out_hi, add_bias):
        def do_block(goff, loff, sz):
            pltpu.sync_copy(acc_sh.at[pl.ds(goff, sz)], scl_v.at[pl.ds(0, sz)])

            def row(r, carry):
                s = jnp.full((16,), inv_v[pl.ds(loff + r, 16)][0], jnp.float32)
                for g in range(FH // 16):
                    v = scl_v[r, pl.ds(g * 16, 16)] * s
                    if add_bias:
                        v = v + bias_v[0, pl.ds(g * 16, 16)]
                    scl_v[r, pl.ds(g * 16, 16)] = v
                return carry
            lax.fori_loop(0, sz, row, 0)

            @pl.when(cid == 0)
            def _wlo():
                pltpu.sync_copy(scl_v.at[pl.ds(0, sz)],
                                out_lo.at[pl.ds(goff, sz)])

            @pl.when(cid == 1)
            def _whi():
                pltpu.sync_copy(scl_v.at[pl.ds(0, sz)],
                                out_hi.at[pl.ds(goff, sz)])
        for_stripe_blocks(do_block)

    plsc.subcore_barrier()

    run_phase(nidx_hbm, nblk_v, eidx_hbm, eblk_v, xlo_hbm, xhi_hbm, True)

    plsc.subcore_barrier()

    compute_inverse(bacc_sh)
    scale_and_write(eolo_hbm, eohi_hbm, False)
    zero_acc()

    plsc.subcore_barrier()

    run_phase(eidx_hbm, eblk_v, nidx_hbm, nblk_v, eolo_hbm, eohi_hbm, False)

    plsc.subcore_barrier()

    compute_inverse(dacc_sh)
    scale_and_write(ylo_hbm, yhi_hbm, True)


def _matmul_body(x_ref, w_ref, lo_ref, hi_ref):
    xw = jnp.dot(x_ref[...], w_ref[...], preferred_element_type=jnp.float32)
    lo_ref[...] = xw[:, :FH]
    hi_ref[...] = xw[:, FH:]


def _tc_matmul(x, w):
    return pl.pallas_call(
        _matmul_body,
        out_shape=(jax.ShapeDtypeStruct((N_NODES, FH), jnp.float32),
                   jax.ShapeDtypeStruct((N_NODES, FH), jnp.float32)),
    )(x, w)


def kernel(x, hyperedge_index, hyperedge_weight, W_lin, bias):
    node_idx = hyperedge_index[0].astype(jnp.int32)
    edge_idx = hyperedge_index[1].astype(jnp.int32)
    nidx = node_idx.reshape(NS, ROWS_PER_TILE, CHUNK)
    eidx = edge_idx.reshape(NS, ROWS_PER_TILE, CHUNK)
    zeros1 = jnp.zeros((N_NODES,), jnp.float32)
    zeros2 = jnp.zeros((N_NODES, FH), jnp.float32)
    bias_f = bias.astype(jnp.float32)

    xlo, xhi = _tc_matmul(x, W_lin)

    y_lo, y_hi, _, _ = _mega_kernel(
        nidx, eidx, xlo, xhi, hyperedge_weight.astype(jnp.float32),
        zeros1, zeros2, bias_f[None, :FH], bias_f[None, FH:])

    return jnp.concatenate([y_lo, y_hi], axis=1)


# speedup vs baseline: 1.1479x; 1.1479x over previous
import jax
import jax.numpy as jnp
from jax import lax
from jax.experimental import pallas as pl
from jax.experimental.pallas import tpu as pltpu
from jax.experimental.pallas import tpu_sc as plsc

N_NODES = 10000
N_EDGES = 10000
N_INC = 320000
F = 128

NC = 2
NS = 16
CHUNK = 125
ROWS_TOTAL = N_INC // CHUNK
ROWS_PER_TILE = ROWS_TOTAL // (NC * NS)
BLK = 16
NBLK = ROWS_PER_TILE // BLK

STRIPE = 632
STRIPE_LAST_OFF = (NS - 1) * STRIPE
STRIPE_LAST = N_NODES - STRIPE_LAST_OFF

_mesh = plsc.VectorSubcoreMesh(core_axis_name="c", subcore_axis_name="s")


def _build_row_phase(with_degrees):
    outs = (jax.ShapeDtypeStruct((NC, N_NODES, F), jnp.float32),)
    scratch = [
        pltpu.VMEM((2, BLK, CHUNK), jnp.int32),
        pltpu.VMEM((2, BLK, CHUNK), jnp.int32),
        pltpu.VMEM((2, CHUNK, F), jnp.float32),
        pltpu.VMEM_SHARED((N_NODES, F), jnp.float32),
        pltpu.SemaphoreType.DMA,
        pltpu.SemaphoreType.DMA,
    ]
    if with_degrees:
        outs = outs + (jax.ShapeDtypeStruct((N_NODES,), jnp.float32),) * 4
        scratch += [
            pltpu.VMEM((BLK, CHUNK), jnp.float32),
            pltpu.VMEM((CHUNK,), jnp.float32),
            pltpu.VMEM_SHARED((N_NODES,), jnp.float32),
            pltpu.VMEM_SHARED((N_NODES,), jnp.float32),
            pltpu.SemaphoreType.DMA,
            pltpu.SemaphoreType.DMA,
            pltpu.SemaphoreType.DMA,
        ]

    def body(*refs):
        if with_degrees:
            (srcidx_hbm, dstidx_hbm, table_hbm, zeros2_hbm, w_hbm, zeros1_hbm,
             out_hbm, d0_out, d1_out, b0_out, b1_out,
             sidx_v, didx_v, rows_v, acc_sh, semg, semi,
             wval_v, ones_v, dacc_sh, bacc_sh, semw, semd, semb) = refs
        else:
            (srcidx_hbm, dstidx_hbm, table_hbm, zeros2_hbm, out_hbm,
             sidx_v, didx_v, rows_v, acc_sh, semg, semi) = refs

        cid = lax.axis_index("c")
        sid = lax.axis_index("s")
        wid = cid * NS + sid

        @pl.when(sid < NS - 1)
        def _zero_a():
            off = pl.multiple_of(sid * STRIPE, 8)
            pltpu.sync_copy(zeros2_hbm.at[pl.ds(off, STRIPE)],
                            acc_sh.at[pl.ds(off, STRIPE)])

        @pl.when(sid == NS - 1)
        def _zero_b():
            pltpu.sync_copy(zeros2_hbm.at[pl.ds(STRIPE_LAST_OFF, STRIPE_LAST)],
                            acc_sh.at[pl.ds(STRIPE_LAST_OFF, STRIPE_LAST)])

        if with_degrees:
            @pl.when(sid == 0)
            def _zero_d():
                pltpu.sync_copy(zeros1_hbm, dacc_sh)

            @pl.when(sid == 1)
            def _zero_bdeg():
                pltpu.sync_copy(zeros1_hbm, bacc_sh)

            for i in range((CHUNK + 15) // 16):
                ones_v[pl.ds(min(16 * i, CHUNK - 16), 16)] = jnp.full(
                    (16,), 1.0, jnp.float32)

        pltpu.async_copy(srcidx_hbm.at[wid, pl.ds(0, BLK)], sidx_v.at[0], semi)
        pltpu.async_copy(dstidx_hbm.at[wid, pl.ds(0, BLK)], didx_v.at[0], semi)

        plsc.subcore_barrier()

        def outer(b, carry):
            pb = b % 2
            pltpu.make_async_copy(srcidx_hbm.at[wid, pl.ds(0, BLK)],
                                  sidx_v.at[pb], semi).wait()
            pltpu.make_async_copy(dstidx_hbm.at[wid, pl.ds(0, BLK)],
                                  didx_v.at[pb], semi).wait()

            @pl.when(b < NBLK - 1)
            def _fire_next_block():
                off = pl.multiple_of((b + 1) * BLK, 8)
                pltpu.async_copy(srcidx_hbm.at[wid, pl.ds(off, BLK)],
                                 sidx_v.at[(b + 1) % 2], semi)
                pltpu.async_copy(dstidx_hbm.at[wid, pl.ds(off, BLK)],
                                 didx_v.at[(b + 1) % 2], semi)

            pltpu.async_copy(table_hbm.at[sidx_v.at[pb, 0]], rows_v.at[0],
                             semg)
            if with_degrees:
                pltpu.async_copy(w_hbm.at[didx_v.at[pb, 0]], wval_v.at[0],
                                 semw)

            def inner(j, c2):
                @pl.when(j < BLK - 1)
                def _fire_next():
                    pltpu.async_copy(table_hbm.at[sidx_v.at[pb, j + 1]],
                                     rows_v.at[(j + 1) % 2], semg)
                    if with_degrees:
                        pltpu.async_copy(w_hbm.at[didx_v.at[pb, j + 1]],
                                         wval_v.at[j + 1], semw)
                pltpu.make_async_copy(table_hbm.at[sidx_v.at[pb, 0]],
                                      rows_v.at[j % 2], semg).wait()
                pltpu.sync_copy(rows_v.at[j % 2],
                                acc_sh.at[didx_v.at[pb, j]], add=True)
                if with_degrees:
                    pltpu.make_async_copy(w_hbm.at[didx_v.at[pb, 0]],
                                          wval_v.at[0], semw).wait()
                    pltpu.async_copy(wval_v.at[j],
                                     dacc_sh.at[sidx_v.at[pb, j]], semd,
                                     add=True)
                    pltpu.async_copy(ones_v, bacc_sh.at[didx_v.at[pb, j]],
                                     semb, add=True)
                return c2
            lax.fori_loop(0, BLK, inner, 0)

            if with_degrees:
                def draind(j, c3):
                    pltpu.make_async_copy(
                        wval_v.at[0], dacc_sh.at[sidx_v.at[0, 0]],
                        semd).wait()
                    return c3
                lax.fori_loop(0, BLK, draind, 0)
            return carry
        lax.fori_loop(0, NBLK, outer, 0)

        if with_degrees:
            def drainb(j, c4):
                pltpu.make_async_copy(ones_v, bacc_sh.at[didx_v.at[0, 0]],
                                      semb).wait()
                return c4
            lax.fori_loop(0, ROWS_PER_TILE, drainb, 0)

        plsc.subcore_barrier()

        @pl.when(sid < NS - 1)
        def _write_a():
            off = pl.multiple_of(sid * STRIPE, 8)
            pltpu.sync_copy(acc_sh.at[pl.ds(off, STRIPE)],
                            out_hbm.at[cid, pl.ds(off, STRIPE)])

        @pl.when(sid == NS - 1)
        def _write_b():
            pltpu.sync_copy(acc_sh.at[pl.ds(STRIPE_LAST_OFF, STRIPE_LAST)],
                            out_hbm.at[cid, pl.ds(STRIPE_LAST_OFF,
                                                  STRIPE_LAST)])

        if with_degrees:
            @pl.when(jnp.logical_and(sid == 0, cid == 0))
            def _write_d0():
                pltpu.sync_copy(dacc_sh, d0_out)

            @pl.when(jnp.logical_and(sid == 0, cid == 1))
            def _write_d1():
                pltpu.sync_copy(dacc_sh, d1_out)

            @pl.when(jnp.logical_and(sid == 1, cid == 0))
            def _write_b0():
                pltpu.sync_copy(bacc_sh, b0_out)

            @pl.when(jnp.logical_and(sid == 1, cid == 1))
            def _write_b1():
                pltpu.sync_copy(bacc_sh, b1_out)

    return pl.kernel(body, out_type=outs, mesh=_mesh, scratch_types=scratch)


_row_phase_deg = _build_row_phase(True)
_row_phase = _build_row_phase(False)


_TC_BLK = 1000
_TC_GRID = N_NODES // _TC_BLK


def _matmul_body(x_ref, w_ref, o_ref):
    o_ref[...] = jnp.dot(x_ref[...], w_ref[...],
                         preferred_element_type=jnp.float32)


def _tc_matmul(x, w):
    return pl.pallas_call(
        _matmul_body,
        grid=(_TC_GRID,),
        in_specs=[pl.BlockSpec((_TC_BLK, F), lambda i: (i, 0)),
                  pl.BlockSpec((F, F), lambda i: (0, 0))],
        out_specs=pl.BlockSpec((_TC_BLK, F), lambda i: (i, 0)),
        out_shape=jax.ShapeDtypeStruct((N_NODES, F), jnp.float32),
    )(x, w)


def _combine_body(p_ref, dega_ref, degb_ref, bias_ref, o_ref):
    d = dega_ref[...] + degb_ref[...]
    inv = jnp.where(d > 0, 1.0 / jnp.where(d > 0, d, 1.0), 0.0)
    o_ref[...] = (p_ref[0] + p_ref[1]) * inv + bias_ref[...]


def _tc_combine(partials, dega, degb, bias_row):
    return pl.pallas_call(
        _combine_body,
        grid=(_TC_GRID,),
        in_specs=[pl.BlockSpec((NC, _TC_BLK, F), lambda i: (0, i, 0)),
                  pl.BlockSpec((_TC_BLK, 1), lambda i: (i, 0)),
                  pl.BlockSpec((_TC_BLK, 1), lambda i: (i, 0)),
                  pl.BlockSpec((1, F), lambda i: (0, 0))],
        out_specs=pl.BlockSpec((_TC_BLK, F), lambda i: (i, 0)),
        out_shape=jax.ShapeDtypeStruct((N_NODES, F), jnp.float32),
    )(partials, dega, degb, bias_row)


def kernel(x, hyperedge_index, hyperedge_weight, W_lin, bias):
    node_idx = hyperedge_index[0].astype(jnp.int32)
    edge_idx = hyperedge_index[1].astype(jnp.int32)
    nidx = node_idx.reshape(NC * NS, ROWS_PER_TILE, CHUNK)
    eidx = edge_idx.reshape(NC * NS, ROWS_PER_TILE, CHUNK)
    zeros1 = jnp.zeros((N_NODES,), jnp.float32)
    zeros2 = jnp.zeros((N_NODES, F), jnp.float32)
    zero_bias = jnp.zeros((1, F), jnp.float32)

    xl = _tc_matmul(x, W_lin)

    p, d0, d1, b0, b1 = _row_phase_deg(
        nidx, eidx, xl, zeros2, hyperedge_weight.astype(jnp.float32), zeros1)
    edge_out = _tc_combine(p, b0[:, None], b1[:, None], zero_bias)

    (q,) = _row_phase(eidx, nidx, edge_out, zeros2)
    return _tc_combine(q, d0[:, None], d1[:, None],
                       bias[None, :].astype(jnp.float32))


# speedup vs baseline: 1.1714x; 1.0204x over previous
import jax
import jax.numpy as jnp
from jax import lax
from jax.experimental import pallas as pl
from jax.experimental.pallas import tpu as pltpu
from jax.experimental.pallas import tpu_sc as plsc

N_NODES = 10000
N_EDGES = 10000
N_INC = 320000
F = 128

NC = 2
NS = 16
CHUNK = 125
ROWS_TOTAL = N_INC // CHUNK
ROWS_PER_TILE = ROWS_TOTAL // (NC * NS)
BLK = 16
NBLK = ROWS_PER_TILE // BLK

STRIPE = 632
STRIPE_LAST_OFF = (NS - 1) * STRIPE
STRIPE_LAST = N_NODES - STRIPE_LAST_OFF

_mesh = plsc.VectorSubcoreMesh(core_axis_name="c", subcore_axis_name="s")


def _build_row_phase(with_degrees):
    outs = (jax.ShapeDtypeStruct((NC, N_NODES, F), jnp.float32),)
    scratch = [
        pltpu.VMEM((2, BLK, CHUNK), jnp.int32),
        pltpu.VMEM((2, BLK, CHUNK), jnp.int32),
        pltpu.VMEM((2, CHUNK, F), jnp.float32),
        pltpu.VMEM_SHARED((N_NODES, F), jnp.float32),
        pltpu.SemaphoreType.DMA,
        pltpu.SemaphoreType.DMA,
    ]
    if with_degrees:
        outs = outs + (jax.ShapeDtypeStruct((N_NODES,), jnp.float32),) * 4
        scratch += [
            pltpu.VMEM((BLK, CHUNK), jnp.float32),
            pltpu.VMEM((CHUNK,), jnp.float32),
            pltpu.VMEM_SHARED((N_NODES,), jnp.float32),
            pltpu.VMEM_SHARED((N_NODES,), jnp.float32),
            pltpu.SemaphoreType.DMA,
            pltpu.SemaphoreType.DMA,
            pltpu.SemaphoreType.DMA,
        ]

    def body(*refs):
        if with_degrees:
            (srcidx_hbm, dstidx_hbm, table_hbm, zeros2_hbm, w_hbm, zeros1_hbm,
             out_hbm, d0_out, d1_out, b0_out, b1_out,
             sidx_v, didx_v, rows_v, acc_sh, semg, semi,
             wval_v, ones_v, dacc_sh, bacc_sh, semw, semd, semb) = refs
        else:
            (srcidx_hbm, dstidx_hbm, table_hbm, zeros2_hbm, out_hbm,
             sidx_v, didx_v, rows_v, acc_sh, semg, semi) = refs

        cid = lax.axis_index("c")
        sid = lax.axis_index("s")
        wid = cid * NS + sid

        @pl.when(sid < NS - 1)
        def _zero_a():
            off = pl.multiple_of(sid * STRIPE, 8)
            pltpu.sync_copy(zeros2_hbm.at[pl.ds(off, STRIPE)],
                            acc_sh.at[pl.ds(off, STRIPE)])

        @pl.when(sid == NS - 1)
        def _zero_b():
            pltpu.sync_copy(zeros2_hbm.at[pl.ds(STRIPE_LAST_OFF, STRIPE_LAST)],
                            acc_sh.at[pl.ds(STRIPE_LAST_OFF, STRIPE_LAST)])

        if with_degrees:
            @pl.when(sid == 0)
            def _zero_d():
                pltpu.sync_copy(zeros1_hbm, dacc_sh)

            @pl.when(sid == 1)
            def _zero_bdeg():
                pltpu.sync_copy(zeros1_hbm, bacc_sh)

            for i in range((CHUNK + 15) // 16):
                ones_v[pl.ds(min(16 * i, CHUNK - 16), 16)] = jnp.full(
                    (16,), 1.0, jnp.float32)

        pltpu.async_copy(srcidx_hbm.at[wid, pl.ds(0, BLK)], sidx_v.at[0], semi)
        pltpu.async_copy(dstidx_hbm.at[wid, pl.ds(0, BLK)], didx_v.at[0], semi)

        plsc.subcore_barrier()

        def outer(b, carry):
            pb = b % 2
            pltpu.make_async_copy(srcidx_hbm.at[wid, pl.ds(0, BLK)],
                                  sidx_v.at[pb], semi).wait()
            pltpu.make_async_copy(dstidx_hbm.at[wid, pl.ds(0, BLK)],
                                  didx_v.at[pb], semi).wait()

            @pl.when(b < NBLK - 1)
            def _fire_next_block():
                off = pl.multiple_of((b + 1) * BLK, 8)
                pltpu.async_copy(srcidx_hbm.at[wid, pl.ds(off, BLK)],
                                 sidx_v.at[(b + 1) % 2], semi)
                pltpu.async_copy(dstidx_hbm.at[wid, pl.ds(off, BLK)],
                                 didx_v.at[(b + 1) % 2], semi)

            pltpu.async_copy(table_hbm.at[sidx_v.at[pb, 0]], rows_v.at[0],
                             semg)
            if with_degrees:
                pltpu.async_copy(w_hbm.at[didx_v.at[pb, 0]], wval_v.at[0],
                                 semw)

            def inner(j, c2):
                @pl.when(j < BLK - 1)
                def _fire_next():
                    pltpu.async_copy(table_hbm.at[sidx_v.at[pb, j + 1]],
                                     rows_v.at[(j + 1) % 2], semg)
                    if with_degrees:
                        pltpu.async_copy(w_hbm.at[didx_v.at[pb, j + 1]],
                                         wval_v.at[j + 1], semw)
                pltpu.make_async_copy(table_hbm.at[sidx_v.at[pb, 0]],
                                      rows_v.at[j % 2], semg).wait()
                pltpu.sync_copy(rows_v.at[j % 2],
                                acc_sh.at[didx_v.at[pb, j]], add=True)
                if with_degrees:
                    pltpu.make_async_copy(w_hbm.at[didx_v.at[pb, 0]],
                                          wval_v.at[0], semw).wait()
                    pltpu.async_copy(wval_v.at[j],
                                     dacc_sh.at[sidx_v.at[pb, j]], semd,
                                     add=True)
                    pltpu.async_copy(ones_v, bacc_sh.at[didx_v.at[pb, j]],
                                     semb, add=True)
                return c2
            lax.fori_loop(0, BLK, inner, 0)

            if with_degrees:
                def draind(j, c3):
                    pltpu.make_async_copy(
                        wval_v.at[0], dacc_sh.at[sidx_v.at[0, 0]],
                        semd).wait()
                    return c3
                lax.fori_loop(0, BLK, draind, 0)
            return carry
        lax.fori_loop(0, NBLK, outer, 0)

        if with_degrees:
            def drainb(j, c4):
                pltpu.make_async_copy(ones_v, bacc_sh.at[didx_v.at[0, 0]],
                                      semb).wait()
                return c4
            lax.fori_loop(0, ROWS_PER_TILE, drainb, 0)

        plsc.subcore_barrier()

        @pl.when(sid < NS - 1)
        def _write_a():
            off = pl.multiple_of(sid * STRIPE, 8)
            pltpu.sync_copy(acc_sh.at[pl.ds(off, STRIPE)],
                            out_hbm.at[cid, pl.ds(off, STRIPE)])

        @pl.when(sid == NS - 1)
        def _write_b():
            pltpu.sync_copy(acc_sh.at[pl.ds(STRIPE_LAST_OFF, STRIPE_LAST)],
                            out_hbm.at[cid, pl.ds(STRIPE_LAST_OFF,
                                                  STRIPE_LAST)])

        if with_degrees:
            @pl.when(jnp.logical_and(sid == 0, cid == 0))
            def _write_d0():
                pltpu.sync_copy(dacc_sh, d0_out)

            @pl.when(jnp.logical_and(sid == 0, cid == 1))
            def _write_d1():
                pltpu.sync_copy(dacc_sh, d1_out)

            @pl.when(jnp.logical_and(sid == 1, cid == 0))
            def _write_b0():
                pltpu.sync_copy(bacc_sh, b0_out)

            @pl.when(jnp.logical_and(sid == 1, cid == 1))
            def _write_b1():
                pltpu.sync_copy(bacc_sh, b1_out)

    return pl.kernel(body, out_type=outs, mesh=_mesh, scratch_types=scratch)


_row_phase_deg = _build_row_phase(True)
_row_phase = _build_row_phase(False)


_TC_BLK = 2000
_TC_GRID = N_NODES // _TC_BLK


def _matmul_body(x_ref, w_ref, o_ref):
    o_ref[...] = jnp.dot(x_ref[...], w_ref[...],
                         preferred_element_type=jnp.float32)


def _tc_matmul(x, w):
    return pl.pallas_call(
        _matmul_body,
        grid=(_TC_GRID,),
        in_specs=[pl.BlockSpec((_TC_BLK, F), lambda i: (i, 0)),
                  pl.BlockSpec((F, F), lambda i: (0, 0))],
        out_specs=pl.BlockSpec((_TC_BLK, F), lambda i: (i, 0)),
        out_shape=jax.ShapeDtypeStruct((N_NODES, F), jnp.float32),
    )(x, w)


def _combine_body(p_ref, dega_ref, degb_ref, bias_ref, o_ref):
    d = dega_ref[...] + degb_ref[...]
    inv = jnp.where(d > 0, 1.0 / jnp.where(d > 0, d, 1.0), 0.0)
    o_ref[...] = (p_ref[0] + p_ref[1]) * inv + bias_ref[...]


def _tc_combine(partials, dega, degb, bias_row):
    return pl.pallas_call(
        _combine_body,
        grid=(_TC_GRID,),
        in_specs=[pl.BlockSpec((NC, _TC_BLK, F), lambda i: (0, i, 0)),
                  pl.BlockSpec((_TC_BLK, 1), lambda i: (i, 0)),
                  pl.BlockSpec((_TC_BLK, 1), lambda i: (i, 0)),
                  pl.BlockSpec((1, F), lambda i: (0, 0))],
        out_specs=pl.BlockSpec((_TC_BLK, F), lambda i: (i, 0)),
        out_shape=jax.ShapeDtypeStruct((N_NODES, F), jnp.float32),
    )(partials, dega, degb, bias_row)


def kernel(x, hyperedge_index, hyperedge_weight, W_lin, bias):
    node_idx = hyperedge_index[0].astype(jnp.int32)
    edge_idx = hyperedge_index[1].astype(jnp.int32)
    nidx = node_idx.reshape(NC * NS, ROWS_PER_TILE, CHUNK)
    eidx = edge_idx.reshape(NC * NS, ROWS_PER_TILE, CHUNK)
    zeros1 = jnp.zeros((N_NODES,), jnp.float32)
    zeros2 = jnp.zeros((N_NODES, F), jnp.float32)
    zero_bias = jnp.zeros((1, F), jnp.float32)

    xl = _tc_matmul(x, W_lin)

    p, d0, d1, b0, b1 = _row_phase_deg(
        nidx, eidx, xl, zeros2, hyperedge_weight.astype(jnp.float32), zeros1)
    edge_out = _tc_combine(p, b0[:, None], b1[:, None], zero_bias)

    (q,) = _row_phase(eidx, nidx, edge_out, zeros2)
    return _tc_combine(q, d0[:, None], d1[:, None],
                       bias[None, :].astype(jnp.float32))


# speedup vs baseline: 1.1769x; 1.0047x over previous
import jax
import jax.numpy as jnp
from jax import lax
from jax.experimental import pallas as pl
from jax.experimental.pallas import tpu as pltpu
from jax.experimental.pallas import tpu_sc as plsc

N_NODES = 10000
N_EDGES = 10000
N_INC = 320000
F = 128

NC = 2
NS = 16
CHUNK = 125
ROWS_TOTAL = N_INC // CHUNK
ROWS_PER_TILE = ROWS_TOTAL // (NC * NS)
BLK = 16
NBLK = ROWS_PER_TILE // BLK

STRIPE = 632
STRIPE_LAST_OFF = (NS - 1) * STRIPE
STRIPE_LAST = N_NODES - STRIPE_LAST_OFF

_mesh = plsc.VectorSubcoreMesh(core_axis_name="c", subcore_axis_name="s")


def _build_row_phase(with_degrees):
    outs = (jax.ShapeDtypeStruct((NC, N_NODES, F), jnp.float32),)
    scratch = [
        pltpu.VMEM((2, BLK, CHUNK), jnp.int32),
        pltpu.VMEM((2, BLK, CHUNK), jnp.int32),
        pltpu.VMEM((2, CHUNK, F), jnp.float32),
        pltpu.VMEM_SHARED((N_NODES, F), jnp.float32),
        pltpu.SemaphoreType.DMA,
        pltpu.SemaphoreType.DMA,
    ]
    if with_degrees:
        outs = outs + (jax.ShapeDtypeStruct((N_NODES,), jnp.float32),) * 4
        scratch += [
            pltpu.VMEM((BLK, CHUNK), jnp.float32),
            pltpu.VMEM((CHUNK,), jnp.float32),
            pltpu.VMEM_SHARED((N_NODES,), jnp.float32),
            pltpu.VMEM_SHARED((N_NODES,), jnp.float32),
            pltpu.SemaphoreType.DMA,
            pltpu.SemaphoreType.DMA,
            pltpu.SemaphoreType.DMA,
        ]

    def body(*refs):
        if with_degrees:
            (srcidx_hbm, dstidx_hbm, table_hbm, zeros2_hbm, w_hbm, zeros1_hbm,
             out_hbm, d0_out, d1_out, b0_out, b1_out,
             sidx_v, didx_v, rows_v, acc_sh, semg, semi,
             wval_v, ones_v, dacc_sh, bacc_sh, semw, semd, semb) = refs
        else:
            (srcidx_hbm, dstidx_hbm, table_hbm, zeros2_hbm, out_hbm,
             sidx_v, didx_v, rows_v, acc_sh, semg, semi) = refs

        cid = lax.axis_index("c")
        sid = lax.axis_index("s")
        wid = cid * NS + sid

        @pl.when(sid < NS - 1)
        def _zero_a():
            off = pl.multiple_of(sid * STRIPE, 8)
            pltpu.sync_copy(zeros2_hbm.at[pl.ds(off, STRIPE)],
                            acc_sh.at[pl.ds(off, STRIPE)])

        @pl.when(sid == NS - 1)
        def _zero_b():
            pltpu.sync_copy(zeros2_hbm.at[pl.ds(STRIPE_LAST_OFF, STRIPE_LAST)],
                            acc_sh.at[pl.ds(STRIPE_LAST_OFF, STRIPE_LAST)])

        if with_degrees:
            @pl.when(sid == 0)
            def _zero_d():
                pltpu.sync_copy(zeros1_hbm, dacc_sh)

            @pl.when(sid == 1)
            def _zero_bdeg():
                pltpu.sync_copy(zeros1_hbm, bacc_sh)

            for i in range((CHUNK + 15) // 16):
                ones_v[pl.ds(min(16 * i, CHUNK - 16), 16)] = jnp.full(
                    (16,), 1.0, jnp.float32)

        pltpu.async_copy(srcidx_hbm.at[wid, pl.ds(0, BLK)], sidx_v.at[0], semi)
        pltpu.async_copy(dstidx_hbm.at[wid, pl.ds(0, BLK)], didx_v.at[0], semi)

        plsc.subcore_barrier()

        def outer(b, carry):
            pb = b % 2
            pltpu.make_async_copy(srcidx_hbm.at[wid, pl.ds(0, BLK)],
                                  sidx_v.at[pb], semi).wait()
            pltpu.make_async_copy(dstidx_hbm.at[wid, pl.ds(0, BLK)],
                                  didx_v.at[pb], semi).wait()

            @pl.when(b < NBLK - 1)
            def _fire_next_block():
                off = pl.multiple_of((b + 1) * BLK, 8)
                pltpu.async_copy(srcidx_hbm.at[wid, pl.ds(off, BLK)],
                                 sidx_v.at[(b + 1) % 2], semi)
                pltpu.async_copy(dstidx_hbm.at[wid, pl.ds(off, BLK)],
                                 didx_v.at[(b + 1) % 2], semi)

            pltpu.async_copy(table_hbm.at[sidx_v.at[pb, 0]], rows_v.at[0],
                             semg)
            if with_degrees:
                pltpu.async_copy(w_hbm.at[didx_v.at[pb, 0]], wval_v.at[0],
                                 semw)

            def inner(j, c2):
                @pl.when(j < BLK - 1)
                def _fire_next():
                    pltpu.async_copy(table_hbm.at[sidx_v.at[pb, j + 1]],
                                     rows_v.at[(j + 1) % 2], semg)
                    if with_degrees:
                        pltpu.async_copy(w_hbm.at[didx_v.at[pb, j + 1]],
                                         wval_v.at[j + 1], semw)
                pltpu.make_async_copy(table_hbm.at[sidx_v.at[pb, 0]],
                                      rows_v.at[j % 2], semg).wait()
                pltpu.sync_copy(rows_v.at[j % 2],
                                acc_sh.at[didx_v.at[pb, j]], add=True)
                if with_degrees:
                    pltpu.make_async_copy(w_hbm.at[didx_v.at[pb, 0]],
                                          wval_v.at[0], semw).wait()
                    pltpu.async_copy(wval_v.at[j],
                                     dacc_sh.at[sidx_v.at[pb, j]], semd,
                                     add=True)
                    pltpu.async_copy(ones_v, bacc_sh.at[didx_v.at[pb, j]],
                                     semb, add=True)
                return c2
            lax.fori_loop(0, BLK, inner, 0)

            if with_degrees:
                def draind(j, c3):
                    pltpu.make_async_copy(
                        wval_v.at[0], dacc_sh.at[sidx_v.at[0, 0]],
                        semd).wait()
                    return c3
                lax.fori_loop(0, BLK, draind, 0)
            return carry
        lax.fori_loop(0, NBLK, outer, 0)

        if with_degrees:
            def drainb(j, c4):
                pltpu.make_async_copy(ones_v, bacc_sh.at[didx_v.at[0, 0]],
                                      semb).wait()
                return c4
            lax.fori_loop(0, ROWS_PER_TILE, drainb, 0)

        plsc.subcore_barrier()

        @pl.when(sid < NS - 1)
        def _write_a():
            off = pl.multiple_of(sid * STRIPE, 8)
            pltpu.sync_copy(acc_sh.at[pl.ds(off, STRIPE)],
                            out_hbm.at[cid, pl.ds(off, STRIPE)])

        @pl.when(sid == NS - 1)
        def _write_b():
            pltpu.sync_copy(acc_sh.at[pl.ds(STRIPE_LAST_OFF, STRIPE_LAST)],
                            out_hbm.at[cid, pl.ds(STRIPE_LAST_OFF,
                                                  STRIPE_LAST)])

        if with_degrees:
            @pl.when(jnp.logical_and(sid == 0, cid == 0))
            def _write_d0():
                pltpu.sync_copy(dacc_sh, d0_out)

            @pl.when(jnp.logical_and(sid == 0, cid == 1))
            def _write_d1():
                pltpu.sync_copy(dacc_sh, d1_out)

            @pl.when(jnp.logical_and(sid == 1, cid == 0))
            def _write_b0():
                pltpu.sync_copy(bacc_sh, b0_out)

            @pl.when(jnp.logical_and(sid == 1, cid == 1))
            def _write_b1():
                pltpu.sync_copy(bacc_sh, b1_out)

    return pl.kernel(
        body, out_type=outs, mesh=_mesh, scratch_types=scratch,
        compiler_params=pltpu.CompilerParams(use_tc_tiling_on_sc=False))


_row_phase_deg = _build_row_phase(True)
_row_phase = _build_row_phase(False)


def _matmul_body(x_ref, w_ref, o_ref):
    o_ref[...] = jnp.dot(x_ref[...], w_ref[...],
                         preferred_element_type=jnp.float32)


def _tc_matmul(x, w):
    return pl.pallas_call(
        _matmul_body,
        out_shape=jax.ShapeDtypeStruct((N_NODES, F), jnp.float32),
    )(x, w)


def _combine_body(p_ref, dega_ref, degb_ref, bias_ref, o_ref):
    d = dega_ref[...] + degb_ref[...]
    inv = jnp.where(d > 0, 1.0 / jnp.where(d > 0, d, 1.0), 0.0)
    o_ref[...] = (p_ref[0] + p_ref[1]) * inv + bias_ref[...]


def _tc_combine(partials, dega, degb, bias_row):
    return pl.pallas_call(
        _combine_body,
        out_shape=jax.ShapeDtypeStruct((N_NODES, F), jnp.float32),
    )(partials, dega, degb, bias_row)


def kernel(x, hyperedge_index, hyperedge_weight, W_lin, bias):
    node_idx = hyperedge_index[0].astype(jnp.int32)
    edge_idx = hyperedge_index[1].astype(jnp.int32)
    nidx = node_idx.reshape(NC * NS, ROWS_PER_TILE, CHUNK)
    eidx = edge_idx.reshape(NC * NS, ROWS_PER_TILE, CHUNK)
    zeros1 = jnp.zeros((N_NODES,), jnp.float32)
    zeros2 = jnp.zeros((N_NODES, F), jnp.float32)
    zero_bias = jnp.zeros((1, F), jnp.float32)

    xl = _tc_matmul(x, W_lin)

    p, d0, d1, b0, b1 = _row_phase_deg(
        nidx, eidx, xl, zeros2, hyperedge_weight.astype(jnp.float32), zeros1)
    edge_out = _tc_combine(p, b0[:, None], b1[:, None], zero_bias)

    (q,) = _row_phase(eidx, nidx, edge_out, zeros2)
    return _tc_combine(q, d0[:, None], d1[:, None],
                       bias[None, :].astype(jnp.float32))
